# baseline (speedup 1.0000x reference)
import functools

import jax
import jax.numpy as jnp
from jax import lax
from jax.experimental import pallas as pl
from jax.experimental.pallas import tpu as pltpu
from jax.experimental.pallas import tpu_sc as plsc

N = 10000
NP = 10240
E = 320000
D = 128

NC = 2
NS = 16
NW = NC * NS
CHUNK = 80
E_PER_W = E // NW
N_CHUNKS = E_PER_W // CHUNK
N_CHUNKS_P = 126
N_CHUNKS_A = 128
PH = 8
ROWS_PER_TILE = NP // NS
ZB = 40


def _mm_xt(a, w):
    return lax.dot_general(a, w, (((1,), (1,)), ((), ())),
                           preferred_element_type=jnp.float32)


def _tc1_body(x_ref, wl_ref, wr_ref, b_ref, y_ref, z_ref):
    xb = x_ref[...]
    y_ref[...] = _mm_xt(xb, wl_ref[...])
    z_ref[...] = _mm_xt(xb, wr_ref[...]) + b_ref[...]


def _tc2_body(acc_ref, cnt_ref, z_ref, wl_ref, wr_ref, b_ref, y_ref, z2_ref):
    a = acc_ref[0, :, :] + acc_ref[1, :, :]
    c = cnt_ref[0, :, 0:1] + cnt_ref[1, :, 0:1]
    inv = 1.0 / jnp.maximum(c, 1.0)
    h = jnp.maximum(a * inv + z_ref[...], 0.0)
    y_ref[...] = _mm_xt(h, wl_ref[...])
    z2_ref[...] = _mm_xt(h, wr_ref[...]) + b_ref[...]


def _tc3_body(acc_ref, cnt_ref, z_ref, out_ref):
    a = acc_ref[0, :, :] + acc_ref[1, :, :]
    c = cnt_ref[0, :, 0:1] + cnt_ref[1, :, 0:1]
    inv = 1.0 / jnp.maximum(c, 1.0)
    out_ref[...] = a * inv + z_ref[...]


_BM = 1024


def _tc1(x, wl, wr, b):
    grid = (NP // _BM,)
    return pl.pallas_call(
        _tc1_body,
        grid=grid,
        in_specs=[
            pl.BlockSpec((_BM, D), lambda i: (i, 0)),
            pl.BlockSpec((D, D), lambda i: (0, 0)),
            pl.BlockSpec((D, D), lambda i: (0, 0)),
            pl.BlockSpec((1, D), lambda i: (0, 0)),
        ],
        out_specs=[
            pl.BlockSpec((_BM, D), lambda i: (i, 0)),
            pl.BlockSpec((_BM, D), lambda i: (i, 0)),
        ],
        out_shape=[
            jax.ShapeDtypeStruct((NP, D), jnp.float32),
            jax.ShapeDtypeStruct((NP, D), jnp.float32),
        ],
    )(x, wl, wr, b.reshape(1, D))


def _tc2(acc, cnt, z, wl, wr, b):
    grid = (NP // _BM,)
    return pl.pallas_call(
        _tc2_body,
        grid=grid,
        in_specs=[
            pl.BlockSpec((NC, _BM, D), lambda i: (0, i, 0)),
            pl.BlockSpec((NC, _BM, D), lambda i: (0, i, 0)),
            pl.BlockSpec((_BM, D), lambda i: (i, 0)),
            pl.BlockSpec((D, D), lambda i: (0, 0)),
            pl.BlockSpec((D, D), lambda i: (0, 0)),
            pl.BlockSpec((1, D), lambda i: (0, 0)),
        ],
        out_specs=[
            pl.BlockSpec((_BM, D), lambda i: (i, 0)),
            pl.BlockSpec((_BM, D), lambda i: (i, 0)),
        ],
        out_shape=[
            jax.ShapeDtypeStruct((NP, D), jnp.float32),
            jax.ShapeDtypeStruct((NP, D), jnp.float32),
        ],
    )(acc, cnt, z, wl, wr, b.reshape(1, D))


def _tc3(acc, cnt, z):
    grid = (NP // _BM,)
    return pl.pallas_call(
        _tc3_body,
        grid=grid,
        in_specs=[
            pl.BlockSpec((NC, _BM, D), lambda i: (0, i, 0)),
            pl.BlockSpec((NC, _BM, D), lambda i: (0, i, 0)),
            pl.BlockSpec((_BM, D), lambda i: (i, 0)),
        ],
        out_specs=pl.BlockSpec((_BM, D), lambda i: (i, 0)),
        out_shape=jax.ShapeDtypeStruct((NP, D), jnp.float32),
    )(acc, cnt, z)


def _zero_fill(ref, rows, cols):
    z16 = jnp.zeros((16,), jnp.float32)
    for r in range(rows):
        for c in range(cols // 16):
            ref[r, pl.ds(c * 16, 16)] = z16


def _zero_fill3(ref, b, rows, cols):
    z16 = jnp.zeros((16,), jnp.float32)
    for r in range(rows):
        for c in range(cols // 16):
            ref[b, r, pl.ds(c * 16, 16)] = z16


def _sc_agg_body(y_hbm, src_hbm, dst_hbm, acc_out, acc_sh, sA, dA, sB, dB,
                 rows_v, semGA, semGB, semSA, semSB):
    cid = lax.axis_index("c")
    sid = lax.axis_index("s")
    wid = cid * NS + sid
    base_n = sid * ROWS_PER_TILE

    _zero_fill3(rows_v, 0, CHUNK, D)
    for k in range(ROWS_PER_TILE // CHUNK):
        pltpu.sync_copy(rows_v.at[0], acc_sh.at[pl.ds(base_n + k * CHUNK, CHUNK)])

    plsc.subcore_barrier()

    def drain(buf, sem):
        pltpu.make_async_copy(y_hbm.at[pl.ds(0, CHUNK)], rows_v.at[buf],
                              sem).wait()

    pltpu.sync_copy(src_hbm.at[wid, 0], sA)
    pltpu.sync_copy(dst_hbm.at[wid, 0], dA)
    gA = pltpu.async_copy(y_hbm.at[sA], rows_v.at[0], semGA)
    pltpu.sync_copy(src_hbm.at[wid, 1], sB)
    pltpu.sync_copy(dst_hbm.at[wid, 1], dB)
    gB = pltpu.async_copy(y_hbm.at[sB], rows_v.at[1], semGB)
    gA.wait()
    pltpu.async_copy(rows_v.at[0], acc_sh.at[dA], semSA, add=True)
    gB.wait()
    pltpu.async_copy(rows_v.at[1], acc_sh.at[dB], semSB, add=True)

    def step(j, carry):
        c0 = 2 * j
        c1 = 2 * j + 1
        pltpu.sync_copy(src_hbm.at[wid, c0], sA2 := sA)
        drain(0, semSA)
        pltpu.sync_copy(dst_hbm.at[wid, c0], dA)
        g0 = pltpu.async_copy(y_hbm.at[sA2], rows_v.at[0], semGA)
        pltpu.sync_copy(src_hbm.at[wid, c1], sB)
        drain(1, semSB)
        pltpu.sync_copy(dst_hbm.at[wid, c1], dB)
        g1 = pltpu.async_copy(y_hbm.at[sB], rows_v.at[1], semGB)
        g0.wait()
        pltpu.async_copy(rows_v.at[0], acc_sh.at[dA], semSA, add=True)
        g1.wait()
        pltpu.async_copy(rows_v.at[1], acc_sh.at[dB], semSB, add=True)
        return carry

    lax.fori_loop(1, N_CHUNKS_P // 2, step, 0)

    drain(0, semSA)
    drain(1, semSB)

    plsc.subcore_barrier()

    pltpu.sync_copy(acc_sh.at[pl.ds(base_n, ROWS_PER_TILE)],
                    acc_out.at[cid, pl.ds(base_n, ROWS_PER_TILE)])


def _sc_cnt_body(dst_hbm, cnt_out, cnt_sh, dstc_v, ones_v, zb_v, sem):
    cid = lax.axis_index("c")
    sid = lax.axis_index("s")
    wid = cid * NS + sid
    base_n = sid * ROWS_PER_TILE

    _zero_fill(zb_v, ZB, D)
    o16 = jnp.ones((16,), jnp.float32)
    for r in range(CHUNK):
        for c in range(D // 16):
            ones_v[r, pl.ds(c * 16, 16)] = o16
    for k in range(ROWS_PER_TILE // ZB):
        pltpu.sync_copy(zb_v, cnt_sh.at[pl.ds(base_n + k * ZB, ZB)])

    plsc.subcore_barrier()

    def step(j, carry):
        pltpu.sync_copy(dst_hbm.at[wid, j], dstc_v)
        pltpu.sync_copy(ones_v, cnt_sh.at[dstc_v], add=True)
        return carry

    lax.fori_loop(0, N_CHUNKS, step, 0)

    plsc.subcore_barrier()
    pltpu.sync_copy(cnt_sh.at[pl.ds(base_n, ROWS_PER_TILE)],
                    cnt_out.at[cid, pl.ds(base_n, ROWS_PER_TILE)])


def _sc_agg(y, src3, dst3):
    mesh = plsc.VectorSubcoreMesh(core_axis_name="c", subcore_axis_name="s")
    fn = pl.kernel(
        _sc_agg_body,
        out_type=jax.ShapeDtypeStruct((NC, NP, D), jnp.float32),
        mesh=mesh,
        scratch_types=[
            pltpu.VMEM_SHARED((NP, D), jnp.float32),
            pltpu.VMEM((CHUNK,), jnp.int32),
            pltpu.VMEM((CHUNK,), jnp.int32),
            pltpu.VMEM((CHUNK,), jnp.int32),
            pltpu.VMEM((CHUNK,), jnp.int32),
            pltpu.VMEM((2, CHUNK, D), jnp.float32),
            pltpu.SemaphoreType.DMA,
            pltpu.SemaphoreType.DMA,
            pltpu.SemaphoreType.DMA,
            pltpu.SemaphoreType.DMA,
        ],
    )
    return fn(y, src3, dst3)


def _sc_cnt(dst3):
    mesh = plsc.VectorSubcoreMesh(core_axis_name="c", subcore_axis_name="s")
    fn = pl.kernel(
        _sc_cnt_body,
        out_type=jax.ShapeDtypeStruct((NC, NP, D), jnp.float32),
        mesh=mesh,
        scratch_types=[
            pltpu.VMEM_SHARED((NP, D), jnp.float32),
            pltpu.VMEM((CHUNK,), jnp.int32),
            pltpu.VMEM((CHUNK, D), jnp.float32),
            pltpu.VMEM((ZB, D), jnp.float32),
            pltpu.SemaphoreType.DMA,
        ],
    )
    return fn(dst3)


def kernel(x, edge_index, W1l, W1r, b1, W2l, W2r, b2):
    pad = N_CHUNKS_A * CHUNK - E_PER_W
    src = jnp.pad(edge_index[0].astype(jnp.int32).reshape(NW, E_PER_W),
                  ((0, 0), (0, pad))).reshape(NW, N_CHUNKS_A, CHUNK)
    dst = jnp.pad(edge_index[1].astype(jnp.int32).reshape(NW, E_PER_W),
                  ((0, 0), (0, pad)),
                  constant_values=NP - 1).reshape(NW, N_CHUNKS_A, CHUNK)
    xp = jnp.pad(x, ((0, NP - N), (0, 0)))

    cnt = _sc_cnt(dst)
    y1, z1 = _tc1(xp, W1l, W1r, b1)
    acc1 = _sc_agg(y1, src, dst)
    y2, z2 = _tc2(acc1, cnt, z1, W2l, W2r, b2)
    acc2 = _sc_agg(y2, src, dst)
    return _tc3(acc2, cnt, z2)[:N]


# speedup vs baseline: 1.1808x; 1.1808x over previous
import functools

import jax
import jax.numpy as jnp
from jax import lax
from jax.experimental import pallas as pl
from jax.experimental.pallas import tpu as pltpu
from jax.experimental.pallas import tpu_sc as plsc

N = 10000
NP = 10240
E = 320000
D = 128

NC = 2
NS = 16
NW = NC * NS
CHUNK = 80
E_PER_W = E // NW
N_CHUNKS = E_PER_W // CHUNK
N_CHUNKS_P = 126
N_CHUNKS_A = 128
PH = 8
ROWS_PER_TILE = NP // NS
ZB = 40


def _mm_xt(a, w):
    return lax.dot_general(a, w, (((1,), (1,)), ((), ())),
                           preferred_element_type=jnp.float32)


def _tc1_body(x_ref, wl_ref, wr_ref, b_ref, y_ref, z_ref):
    xb = x_ref[...]
    y_ref[...] = _mm_xt(xb, wl_ref[...])
    z_ref[...] = _mm_xt(xb, wr_ref[...]) + b_ref[...]


def _tc2_body(acc_ref, cnt_ref, z_ref, wl_ref, wr_ref, b_ref, y_ref, z2_ref):
    a = acc_ref[0, :, :] + acc_ref[1, :, :]
    c = cnt_ref[0, :, 0:1] + cnt_ref[1, :, 0:1]
    inv = 1.0 / jnp.maximum(c, 1.0)
    h = jnp.maximum(a * inv + z_ref[...], 0.0)
    y_ref[...] = _mm_xt(h, wl_ref[...])
    z2_ref[...] = _mm_xt(h, wr_ref[...]) + b_ref[...]


def _tc3_body(acc_ref, cnt_ref, z_ref, out_ref):
    a = acc_ref[0, :, :] + acc_ref[1, :, :]
    c = cnt_ref[0, :, 0:1] + cnt_ref[1, :, 0:1]
    inv = 1.0 / jnp.maximum(c, 1.0)
    out_ref[...] = a * inv + z_ref[...]


_BM = 1024


def _tc1(x, wl, wr, b):
    grid = (NP // _BM,)
    return pl.pallas_call(
        _tc1_body,
        grid=grid,
        in_specs=[
            pl.BlockSpec((_BM, D), lambda i: (i, 0)),
            pl.BlockSpec((D, D), lambda i: (0, 0)),
            pl.BlockSpec((D, D), lambda i: (0, 0)),
            pl.BlockSpec((1, D), lambda i: (0, 0)),
        ],
        out_specs=[
            pl.BlockSpec((_BM, D), lambda i: (i, 0)),
            pl.BlockSpec((_BM, D), lambda i: (i, 0)),
        ],
        out_shape=[
            jax.ShapeDtypeStruct((NP, D), jnp.float32),
            jax.ShapeDtypeStruct((NP, D), jnp.float32),
        ],
    )(x, wl, wr, b.reshape(1, D))


def _tc2(acc, cnt, z, wl, wr, b):
    grid = (NP // _BM,)
    return pl.pallas_call(
        _tc2_body,
        grid=grid,
        in_specs=[
            pl.BlockSpec((NC, _BM, D), lambda i: (0, i, 0)),
            pl.BlockSpec((NC, _BM, D), lambda i: (0, i, 0)),
            pl.BlockSpec((_BM, D), lambda i: (i, 0)),
            pl.BlockSpec((D, D), lambda i: (0, 0)),
            pl.BlockSpec((D, D), lambda i: (0, 0)),
            pl.BlockSpec((1, D), lambda i: (0, 0)),
        ],
        out_specs=[
            pl.BlockSpec((_BM, D), lambda i: (i, 0)),
            pl.BlockSpec((_BM, D), lambda i: (i, 0)),
        ],
        out_shape=[
            jax.ShapeDtypeStruct((NP, D), jnp.float32),
            jax.ShapeDtypeStruct((NP, D), jnp.float32),
        ],
    )(acc, cnt, z, wl, wr, b.reshape(1, D))


def _tc3(acc, cnt, z):
    grid = (NP // _BM,)
    return pl.pallas_call(
        _tc3_body,
        grid=grid,
        in_specs=[
            pl.BlockSpec((NC, _BM, D), lambda i: (0, i, 0)),
            pl.BlockSpec((NC, _BM, D), lambda i: (0, i, 0)),
            pl.BlockSpec((_BM, D), lambda i: (i, 0)),
        ],
        out_specs=pl.BlockSpec((_BM, D), lambda i: (i, 0)),
        out_shape=jax.ShapeDtypeStruct((NP, D), jnp.float32),
    )(acc, cnt, z)


def _zero_fill(ref, rows, cols):
    z16 = jnp.zeros((16,), jnp.float32)
    for r in range(rows):
        for c in range(cols // 16):
            ref[r, pl.ds(c * 16, 16)] = z16


def _zero_fill3(ref, b, rows, cols):
    z16 = jnp.zeros((16,), jnp.float32)
    for r in range(rows):
        for c in range(cols // 16):
            ref[b, r, pl.ds(c * 16, 16)] = z16


def _sc_agg_body(y_hbm, src_hbm, dst_hbm, acc_out, acc_sh, sA, dA, sB, dB,
                 rows_v, semA, semB):
    cid = lax.axis_index("c")
    sid = lax.axis_index("s")
    wid = cid * NS + sid
    base_n = sid * ROWS_PER_TILE

    _zero_fill3(rows_v, 0, CHUNK, D)
    for k in range(ROWS_PER_TILE // CHUNK):
        pltpu.sync_copy(rows_v.at[0], acc_sh.at[pl.ds(base_n + k * CHUNK, CHUNK)])

    plsc.subcore_barrier()

    pltpu.sync_copy(src_hbm.at[wid, 0], sA)
    pltpu.sync_copy(dst_hbm.at[wid, 0], dA)
    g0 = pltpu.async_copy(y_hbm.at[sA], rows_v.at[0], semA)
    g0.wait()

    def step(j, carry):
        c0 = 2 * j + 1
        c1 = 2 * j + 2
        pltpu.sync_copy(src_hbm.at[wid, c0], sB)
        pltpu.sync_copy(dst_hbm.at[wid, c0], dB)
        gB = pltpu.async_copy(y_hbm.at[sB], rows_v.at[1], semB)
        pltpu.sync_copy(rows_v.at[0], acc_sh.at[dA], add=True)
        pltpu.sync_copy(src_hbm.at[wid, c1], sA)
        pltpu.sync_copy(dst_hbm.at[wid, c1], dA)
        gA = pltpu.async_copy(y_hbm.at[sA], rows_v.at[0], semA)
        gB.wait()
        pltpu.sync_copy(rows_v.at[1], acc_sh.at[dB], add=True)
        gA.wait()
        return carry

    lax.fori_loop(0, (N_CHUNKS - 1) // 2, step, 0)
    pltpu.sync_copy(rows_v.at[0], acc_sh.at[dA], add=True)

    plsc.subcore_barrier()

    pltpu.sync_copy(acc_sh.at[pl.ds(base_n, ROWS_PER_TILE)],
                    acc_out.at[cid, pl.ds(base_n, ROWS_PER_TILE)])


def _sc_cnt_body(dst_hbm, cnt_out, cnt_sh, dstc_v, ones_v, zb_v, sem):
    cid = lax.axis_index("c")
    sid = lax.axis_index("s")
    wid = cid * NS + sid
    base_n = sid * ROWS_PER_TILE

    _zero_fill(zb_v, ZB, D)
    o16 = jnp.ones((16,), jnp.float32)
    for r in range(CHUNK):
        for c in range(D // 16):
            ones_v[r, pl.ds(c * 16, 16)] = o16
    for k in range(ROWS_PER_TILE // ZB):
        pltpu.sync_copy(zb_v, cnt_sh.at[pl.ds(base_n + k * ZB, ZB)])

    plsc.subcore_barrier()

    def step(j, carry):
        pltpu.sync_copy(dst_hbm.at[wid, j], dstc_v)
        pltpu.sync_copy(ones_v, cnt_sh.at[dstc_v], add=True)
        return carry

    lax.fori_loop(0, N_CHUNKS, step, 0)

    plsc.subcore_barrier()
    pltpu.sync_copy(cnt_sh.at[pl.ds(base_n, ROWS_PER_TILE)],
                    cnt_out.at[cid, pl.ds(base_n, ROWS_PER_TILE)])


def _sc_agg(y, src3, dst3):
    mesh = plsc.VectorSubcoreMesh(core_axis_name="c", subcore_axis_name="s")
    fn = pl.kernel(
        _sc_agg_body,
        out_type=jax.ShapeDtypeStruct((NC, NP, D), jnp.float32),
        mesh=mesh,
        scratch_types=[
            pltpu.VMEM_SHARED((NP, D), jnp.float32),
            pltpu.VMEM((CHUNK,), jnp.int32),
            pltpu.VMEM((CHUNK,), jnp.int32),
            pltpu.VMEM((CHUNK,), jnp.int32),
            pltpu.VMEM((CHUNK,), jnp.int32),
            pltpu.VMEM((2, CHUNK, D), jnp.float32),
            pltpu.SemaphoreType.DMA,
            pltpu.SemaphoreType.DMA,
        ],
    )
    return fn(y, src3, dst3)


def _sc_cnt(dst3):
    mesh = plsc.VectorSubcoreMesh(core_axis_name="c", subcore_axis_name="s")
    fn = pl.kernel(
        _sc_cnt_body,
        out_type=jax.ShapeDtypeStruct((NC, NP, D), jnp.float32),
        mesh=mesh,
        scratch_types=[
            pltpu.VMEM_SHARED((NP, D), jnp.float32),
            pltpu.VMEM((CHUNK,), jnp.int32),
            pltpu.VMEM((CHUNK, D), jnp.float32),
            pltpu.VMEM((ZB, D), jnp.float32),
            pltpu.SemaphoreType.DMA,
        ],
    )
    return fn(dst3)


def kernel(x, edge_index, W1l, W1r, b1, W2l, W2r, b2):
    pad = N_CHUNKS_A * CHUNK - E_PER_W
    src = jnp.pad(edge_index[0].astype(jnp.int32).reshape(NW, E_PER_W),
                  ((0, 0), (0, pad))).reshape(NW, N_CHUNKS_A, CHUNK)
    dst = jnp.pad(edge_index[1].astype(jnp.int32).reshape(NW, E_PER_W),
                  ((0, 0), (0, pad)),
                  constant_values=NP - 1).reshape(NW, N_CHUNKS_A, CHUNK)
    xp = jnp.pad(x, ((0, NP - N), (0, 0)))

    cnt = _sc_cnt(dst)
    y1, z1 = _tc1(xp, W1l, W1r, b1)
    acc1 = _sc_agg(y1, src, dst)
    y2, z2 = _tc2(acc1, cnt, z1, W2l, W2r, b2)
    acc2 = _sc_agg(y2, src, dst)
    return _tc3(acc2, cnt, z2)[:N]


# speedup vs baseline: 1.1882x; 1.0063x over previous
import functools

import jax
import jax.numpy as jnp
from jax import lax
from jax.experimental import pallas as pl
from jax.experimental.pallas import tpu as pltpu
from jax.experimental.pallas import tpu_sc as plsc

N = 10000
NP = 10240
E = 320000
D = 128

NC = 2
NS = 16
NW = NC * NS
CHUNK = 80
E_PER_W = E // NW
N_CHUNKS = E_PER_W // CHUNK
N_CHUNKS_P = 126
N_CHUNKS_A = 128
PH = 8
ROWS_PER_TILE = NP // NS
ZB = 40


def _mm_xt(a, w):
    return lax.dot_general(a, w, (((1,), (1,)), ((), ())),
                           preferred_element_type=jnp.float32)


def _tc1_body(x_ref, wl_ref, wr_ref, b_ref, y_ref, z_ref):
    xb = x_ref[...]
    y_ref[...] = _mm_xt(xb, wl_ref[...])
    z_ref[...] = _mm_xt(xb, wr_ref[...]) + b_ref[...]


def _tc2_body(acc_ref, cnt_ref, z_ref, wl_ref, wr_ref, b_ref, y_ref, z2_ref):
    a = acc_ref[0, :, :] + acc_ref[1, :, :]
    c = cnt_ref[0, :, 0:1] + cnt_ref[1, :, 0:1]
    inv = 1.0 / jnp.maximum(c, 1.0)
    h = jnp.maximum(a * inv + z_ref[...], 0.0)
    y_ref[...] = _mm_xt(h, wl_ref[...])
    z2_ref[...] = _mm_xt(h, wr_ref[...]) + b_ref[...]


def _tc3_body(acc_ref, cnt_ref, z_ref, out_ref):
    a = acc_ref[0, :, :] + acc_ref[1, :, :]
    c = cnt_ref[0, :, 0:1] + cnt_ref[1, :, 0:1]
    inv = 1.0 / jnp.maximum(c, 1.0)
    out_ref[...] = a * inv + z_ref[...]


_BM = 1024


def _tc1(x, wl, wr, b):
    grid = (NP // _BM,)
    return pl.pallas_call(
        _tc1_body,
        grid=grid,
        in_specs=[
            pl.BlockSpec((_BM, D), lambda i: (i, 0)),
            pl.BlockSpec((D, D), lambda i: (0, 0)),
            pl.BlockSpec((D, D), lambda i: (0, 0)),
            pl.BlockSpec((1, D), lambda i: (0, 0)),
        ],
        out_specs=[
            pl.BlockSpec((_BM, D), lambda i: (i, 0)),
            pl.BlockSpec((_BM, D), lambda i: (i, 0)),
        ],
        out_shape=[
            jax.ShapeDtypeStruct((NP, D), jnp.float32),
            jax.ShapeDtypeStruct((NP, D), jnp.float32),
        ],
    )(x, wl, wr, b.reshape(1, D))


def _tc2(acc, cnt, z, wl, wr, b):
    grid = (NP // _BM,)
    return pl.pallas_call(
        _tc2_body,
        grid=grid,
        in_specs=[
            pl.BlockSpec((NC, _BM, D), lambda i: (0, i, 0)),
            pl.BlockSpec((NC, _BM, D), lambda i: (0, i, 0)),
            pl.BlockSpec((_BM, D), lambda i: (i, 0)),
            pl.BlockSpec((D, D), lambda i: (0, 0)),
            pl.BlockSpec((D, D), lambda i: (0, 0)),
            pl.BlockSpec((1, D), lambda i: (0, 0)),
        ],
        out_specs=[
            pl.BlockSpec((_BM, D), lambda i: (i, 0)),
            pl.BlockSpec((_BM, D), lambda i: (i, 0)),
        ],
        out_shape=[
            jax.ShapeDtypeStruct((NP, D), jnp.float32),
            jax.ShapeDtypeStruct((NP, D), jnp.float32),
        ],
    )(acc, cnt, z, wl, wr, b.reshape(1, D))


def _tc3(acc, cnt, z):
    grid = (NP // _BM,)
    return pl.pallas_call(
        _tc3_body,
        grid=grid,
        in_specs=[
            pl.BlockSpec((NC, _BM, D), lambda i: (0, i, 0)),
            pl.BlockSpec((NC, _BM, D), lambda i: (0, i, 0)),
            pl.BlockSpec((_BM, D), lambda i: (i, 0)),
        ],
        out_specs=pl.BlockSpec((_BM, D), lambda i: (i, 0)),
        out_shape=jax.ShapeDtypeStruct((NP, D), jnp.float32),
    )(acc, cnt, z)


def _zero_fill(ref, rows, cols):
    z16 = jnp.zeros((16,), jnp.float32)
    for r in range(rows):
        for c in range(cols // 16):
            ref[r, pl.ds(c * 16, 16)] = z16


def _zero_fill3(ref, b, rows, cols):
    z16 = jnp.zeros((16,), jnp.float32)
    for r in range(rows):
        for c in range(cols // 16):
            ref[b, r, pl.ds(c * 16, 16)] = z16


def _ones_fill3(ref, b, rows, cols):
    o16 = jnp.ones((16,), jnp.float32)
    for r in range(rows):
        for c in range(cols // 16):
            ref[b, r, pl.ds(c * 16, 16)] = o16


def _zero_acc(rows_v, acc_sh, base_n):
    _zero_fill3(rows_v, 0, CHUNK, D)
    for k in range(ROWS_PER_TILE // CHUNK):
        pltpu.sync_copy(rows_v.at[0], acc_sh.at[pl.ds(base_n + k * CHUNK, CHUNK)])


def _agg_pipeline(y_hbm, src_hbm, dst_hbm, acc_sh, sA, dA, sB, dB, rows_v,
                  semA, semB, wid):
    pltpu.sync_copy(src_hbm.at[wid, 0], sA)
    pltpu.sync_copy(dst_hbm.at[wid, 0], dA)
    g0 = pltpu.async_copy(y_hbm.at[sA], rows_v.at[0], semA)
    g0.wait()

    def step(j, carry):
        c0 = 2 * j + 1
        c1 = 2 * j + 2
        pltpu.sync_copy(src_hbm.at[wid, c0], sB)
        pltpu.sync_copy(dst_hbm.at[wid, c0], dB)
        gB = pltpu.async_copy(y_hbm.at[sB], rows_v.at[1], semB)
        pltpu.sync_copy(rows_v.at[0], acc_sh.at[dA], add=True)
        pltpu.sync_copy(src_hbm.at[wid, c1], sA)
        pltpu.sync_copy(dst_hbm.at[wid, c1], dA)
        gA = pltpu.async_copy(y_hbm.at[sA], rows_v.at[0], semA)
        gB.wait()
        pltpu.sync_copy(rows_v.at[1], acc_sh.at[dB], add=True)
        gA.wait()
        return carry

    lax.fori_loop(0, (N_CHUNKS - 1) // 2, step, 0)
    pltpu.sync_copy(rows_v.at[0], acc_sh.at[dA], add=True)


def _sc_agg_body(with_cnt, y_hbm, src_hbm, dst_hbm, *rest):
    if with_cnt:
        (acc_out, cnt_out, acc_sh, sA, dA, sB, dB, rows_v, semA, semB) = rest
    else:
        (acc_out, acc_sh, sA, dA, sB, dB, rows_v, semA, semB) = rest
        cnt_out = None
    cid = lax.axis_index("c")
    sid = lax.axis_index("s")
    wid = cid * NS + sid
    base_n = sid * ROWS_PER_TILE

    if with_cnt:
        _zero_acc(rows_v, acc_sh, base_n)
        _ones_fill3(rows_v, 1, CHUNK, D)
        plsc.subcore_barrier()

        def cstep(j, carry):
            pltpu.sync_copy(dst_hbm.at[wid, j], dA)
            pltpu.sync_copy(rows_v.at[1], acc_sh.at[dA], add=True)
            return carry

        lax.fori_loop(0, N_CHUNKS, cstep, 0)
        plsc.subcore_barrier()
        pltpu.sync_copy(acc_sh.at[pl.ds(base_n, ROWS_PER_TILE)],
                        cnt_out.at[cid, pl.ds(base_n, ROWS_PER_TILE)])
        plsc.subcore_barrier()

    _zero_acc(rows_v, acc_sh, base_n)
    plsc.subcore_barrier()
    _agg_pipeline(y_hbm, src_hbm, dst_hbm, acc_sh, sA, dA, sB, dB, rows_v,
                  semA, semB, wid)
    plsc.subcore_barrier()
    pltpu.sync_copy(acc_sh.at[pl.ds(base_n, ROWS_PER_TILE)],
                    acc_out.at[cid, pl.ds(base_n, ROWS_PER_TILE)])


def _sc_agg(y, src3, dst3, with_cnt=False):
    mesh = plsc.VectorSubcoreMesh(core_axis_name="c", subcore_axis_name="s")
    out_type = [jax.ShapeDtypeStruct((NC, NP, D), jnp.float32)]
    if with_cnt:
        out_type.append(jax.ShapeDtypeStruct((NC, NP, D), jnp.float32))
    fn = pl.kernel(
        functools.partial(_sc_agg_body, with_cnt),
        out_type=tuple(out_type),
        mesh=mesh,
        scratch_types=[
            pltpu.VMEM_SHARED((NP, D), jnp.float32),
            pltpu.VMEM((CHUNK,), jnp.int32),
            pltpu.VMEM((CHUNK,), jnp.int32),
            pltpu.VMEM((CHUNK,), jnp.int32),
            pltpu.VMEM((CHUNK,), jnp.int32),
            pltpu.VMEM((2, CHUNK, D), jnp.float32),
            pltpu.SemaphoreType.DMA,
            pltpu.SemaphoreType.DMA,
        ],
    )
    out = fn(y, src3, dst3)
    return out if with_cnt else out[0]


def kernel(x, edge_index, W1l, W1r, b1, W2l, W2r, b2):
    pad = N_CHUNKS_A * CHUNK - E_PER_W
    src = jnp.pad(edge_index[0].astype(jnp.int32).reshape(NW, E_PER_W),
                  ((0, 0), (0, pad))).reshape(NW, N_CHUNKS_A, CHUNK)
    dst = jnp.pad(edge_index[1].astype(jnp.int32).reshape(NW, E_PER_W),
                  ((0, 0), (0, pad)),
                  constant_values=NP - 1).reshape(NW, N_CHUNKS_A, CHUNK)
    xp = jnp.pad(x, ((0, NP - N), (0, 0)))

    y1, z1 = _tc1(xp, W1l, W1r, b1)
    acc1, cnt = _sc_agg(y1, src, dst, with_cnt=True)
    y2, z2 = _tc2(acc1, cnt, z1, W2l, W2r, b2)
    acc2 = _sc_agg(y2, src, dst)
    return _tc3(acc2, cnt, z2)[:N]


# speedup vs baseline: 1.2927x; 1.0879x over previous
import functools

import jax
import jax.numpy as jnp
from jax import lax
from jax.experimental import pallas as pl
from jax.experimental.pallas import tpu as pltpu
from jax.experimental.pallas import tpu_sc as plsc

N = 10000
NP = 10240
E = 320000
D = 128

NC = 2
NS = 16
NW = NC * NS
CHUNK = 80
E_PER_W = E // NW
N_CHUNKS = E_PER_W // CHUNK
N_CHUNKS_P = 126
N_CHUNKS_A = 128
PH = 8
ROWS_PER_TILE = NP // NS
ZB = 40


def _mm_xt(a, w):
    return lax.dot_general(a, w, (((1,), (1,)), ((), ())),
                           preferred_element_type=jnp.float32)


def _tc1_body(x_ref, wl_ref, wr_ref, b_ref, y_ref, z_ref):
    xb = x_ref[...]
    y_ref[...] = _mm_xt(xb, wl_ref[...])
    z_ref[...] = _mm_xt(xb, wr_ref[...]) + b_ref[...]


def _tc2_body(acc_ref, cnt_ref, z_ref, wl_ref, wr_ref, b_ref, y_ref, z2_ref):
    a = acc_ref[0, :, :] + acc_ref[1, :, :]
    c = cnt_ref[0, :, 0:1] + cnt_ref[1, :, 0:1]
    inv = 1.0 / jnp.maximum(c, 1.0)
    h = jnp.maximum(a * inv + z_ref[...], 0.0)
    y_ref[...] = _mm_xt(h, wl_ref[...])
    z2_ref[...] = _mm_xt(h, wr_ref[...]) + b_ref[...]


def _tc3_body(acc_ref, cnt_ref, z_ref, out_ref):
    a = acc_ref[0, :, :] + acc_ref[1, :, :]
    c = cnt_ref[0, :, 0:1] + cnt_ref[1, :, 0:1]
    inv = 1.0 / jnp.maximum(c, 1.0)
    out_ref[...] = a * inv + z_ref[...]


_BM = 1024


def _tc1(x, wl, wr, b):
    grid = (NP // _BM,)
    return pl.pallas_call(
        _tc1_body,
        grid=grid,
        in_specs=[
            pl.BlockSpec((_BM, D), lambda i: (i, 0)),
            pl.BlockSpec((D, D), lambda i: (0, 0)),
            pl.BlockSpec((D, D), lambda i: (0, 0)),
            pl.BlockSpec((1, D), lambda i: (0, 0)),
        ],
        out_specs=[
            pl.BlockSpec((_BM, D), lambda i: (i, 0)),
            pl.BlockSpec((_BM, D), lambda i: (i, 0)),
        ],
        out_shape=[
            jax.ShapeDtypeStruct((NP, D), jnp.float32),
            jax.ShapeDtypeStruct((NP, D), jnp.float32),
        ],
    )(x, wl, wr, b.reshape(1, D))


def _tc2(acc, cnt, z, wl, wr, b):
    grid = (NP // _BM,)
    return pl.pallas_call(
        _tc2_body,
        grid=grid,
        in_specs=[
            pl.BlockSpec((NC, _BM, D), lambda i: (0, i, 0)),
            pl.BlockSpec((NC, _BM, D), lambda i: (0, i, 0)),
            pl.BlockSpec((_BM, D), lambda i: (i, 0)),
            pl.BlockSpec((D, D), lambda i: (0, 0)),
            pl.BlockSpec((D, D), lambda i: (0, 0)),
            pl.BlockSpec((1, D), lambda i: (0, 0)),
        ],
        out_specs=[
            pl.BlockSpec((_BM, D), lambda i: (i, 0)),
            pl.BlockSpec((_BM, D), lambda i: (i, 0)),
        ],
        out_shape=[
            jax.ShapeDtypeStruct((NP, D), jnp.float32),
            jax.ShapeDtypeStruct((NP, D), jnp.float32),
        ],
    )(acc, cnt, z, wl, wr, b.reshape(1, D))


def _tc3(acc, cnt, z):
    grid = (NP // _BM,)
    return pl.pallas_call(
        _tc3_body,
        grid=grid,
        in_specs=[
            pl.BlockSpec((NC, _BM, D), lambda i: (0, i, 0)),
            pl.BlockSpec((NC, _BM, D), lambda i: (0, i, 0)),
            pl.BlockSpec((_BM, D), lambda i: (i, 0)),
        ],
        out_specs=pl.BlockSpec((_BM, D), lambda i: (i, 0)),
        out_shape=jax.ShapeDtypeStruct((NP, D), jnp.float32),
    )(acc, cnt, z)


def _zero_fill(ref, rows, cols):
    z16 = jnp.zeros((16,), jnp.float32)
    for r in range(rows):
        for c in range(cols // 16):
            ref[r, pl.ds(c * 16, 16)] = z16


def _zero_fill3(ref, b, rows, cols):
    z16 = jnp.zeros((16,), jnp.float32)
    for r in range(rows):
        for c in range(cols // 16):
            ref[b, r, pl.ds(c * 16, 16)] = z16


def _ones_fill3(ref, b, rows, cols):
    o16 = jnp.ones((16,), jnp.float32)
    for r in range(rows):
        for c in range(cols // 16):
            ref[b, r, pl.ds(c * 16, 16)] = o16


def _zero_acc(rows_v, acc_sh, base_n):
    _zero_fill3(rows_v, 0, CHUNK, D)
    for k in range(ROWS_PER_TILE // CHUNK):
        pltpu.sync_copy(rows_v.at[0], acc_sh.at[pl.ds(base_n + k * CHUNK, CHUNK)])


def _agg_pipeline(y_hbm, src_hbm, dst_hbm, acc_sh, sA, dA, sB, dB, rows_v,
                  semA, semB, wid):
    pltpu.sync_copy(src_hbm.at[wid, 0], sA)
    pltpu.sync_copy(dst_hbm.at[wid, 0], dA)
    g0 = pltpu.async_copy(y_hbm.at[sA], rows_v.at[0], semA)
    g0.wait()

    def step(j, carry):
        c0 = 2 * j + 1
        c1 = 2 * j + 2
        pltpu.sync_copy(src_hbm.at[wid, c0], sB)
        pltpu.sync_copy(dst_hbm.at[wid, c0], dB)
        gB = pltpu.async_copy(y_hbm.at[sB], rows_v.at[1], semB)
        pltpu.sync_copy(rows_v.at[0], acc_sh.at[dA], add=True)
        pltpu.sync_copy(src_hbm.at[wid, c1], sA)
        pltpu.sync_copy(dst_hbm.at[wid, c1], dA)
        gA = pltpu.async_copy(y_hbm.at[sA], rows_v.at[0], semA)
        gB.wait()
        pltpu.sync_copy(rows_v.at[1], acc_sh.at[dB], add=True)
        gA.wait()
        return carry

    lax.fori_loop(0, (N_CHUNKS - 1) // 2, step, 0)
    pltpu.sync_copy(rows_v.at[0], acc_sh.at[dA], add=True)


def _sc_agg_body(with_cnt, y_hbm, src_hbm, dst_hbm, *rest):
    if with_cnt:
        (acc_out, cnt_out, acc_sh, sA, dA, sB, dB, rows_v, semA, semB) = rest
    else:
        (acc_out, acc_sh, sA, dA, sB, dB, rows_v, semA, semB) = rest
        cnt_out = None
    cid = lax.axis_index("c")
    sid = lax.axis_index("s")
    wid = cid * NS + sid
    base_n = sid * ROWS_PER_TILE

    if with_cnt:
        _zero_acc(rows_v, acc_sh, base_n)
        _ones_fill3(rows_v, 1, CHUNK, D)
        plsc.subcore_barrier()

        iA = pltpu.async_copy(dst_hbm.at[wid, 0], dA, semA)
        iA.wait()

        def cstep(j, carry):
            c0 = 2 * j + 1
            c1 = 2 * j + 2
            iB = pltpu.async_copy(dst_hbm.at[wid, c0], dB, semB)
            pltpu.sync_copy(rows_v.at[1], acc_sh.at[dA], add=True)
            iA2 = pltpu.async_copy(dst_hbm.at[wid, c1], dA, semA)
            iB.wait()
            pltpu.sync_copy(rows_v.at[1], acc_sh.at[dB], add=True)
            iA2.wait()
            return carry

        lax.fori_loop(0, (N_CHUNKS - 1) // 2, cstep, 0)
        pltpu.sync_copy(rows_v.at[1], acc_sh.at[dA], add=True)
        plsc.subcore_barrier()
        pltpu.sync_copy(acc_sh.at[pl.ds(base_n, ROWS_PER_TILE)],
                        cnt_out.at[cid, pl.ds(base_n, ROWS_PER_TILE)])
        plsc.subcore_barrier()

    _zero_acc(rows_v, acc_sh, base_n)
    plsc.subcore_barrier()
    _agg_pipeline(y_hbm, src_hbm, dst_hbm, acc_sh, sA, dA, sB, dB, rows_v,
                  semA, semB, wid)
    plsc.subcore_barrier()
    pltpu.sync_copy(acc_sh.at[pl.ds(base_n, ROWS_PER_TILE)],
                    acc_out.at[cid, pl.ds(base_n, ROWS_PER_TILE)])


def _sc_agg(y, src3, dst3, with_cnt=False):
    mesh = plsc.VectorSubcoreMesh(core_axis_name="c", subcore_axis_name="s")
    out_type = [jax.ShapeDtypeStruct((NC, NP, D), jnp.float32)]
    if with_cnt:
        out_type.append(jax.ShapeDtypeStruct((NC, NP, D), jnp.float32))
    fn = pl.kernel(
        functools.partial(_sc_agg_body, with_cnt),
        out_type=tuple(out_type),
        mesh=mesh,
        scratch_types=[
            pltpu.VMEM_SHARED((NP, D), jnp.float32),
            pltpu.VMEM((CHUNK,), jnp.int32),
            pltpu.VMEM((CHUNK,), jnp.int32),
            pltpu.VMEM((CHUNK,), jnp.int32),
            pltpu.VMEM((CHUNK,), jnp.int32),
            pltpu.VMEM((2, CHUNK, D), jnp.float32),
            pltpu.SemaphoreType.DMA,
            pltpu.SemaphoreType.DMA,
        ],
    )
    out = fn(y, src3, dst3)
    return out if with_cnt else out[0]


def kernel(x, edge_index, W1l, W1r, b1, W2l, W2r, b2):
    pad = N_CHUNKS_A * CHUNK - E_PER_W
    src = jnp.pad(edge_index[0].astype(jnp.int32).reshape(NW, E_PER_W),
                  ((0, 0), (0, pad))).reshape(NW, N_CHUNKS_A, CHUNK)
    dst = jnp.pad(edge_index[1].astype(jnp.int32).reshape(NW, E_PER_W),
                  ((0, 0), (0, pad)),
                  constant_values=NP - 1).reshape(NW, N_CHUNKS_A, CHUNK)
    xp = jnp.pad(x, ((0, NP - N), (0, 0)))

    y1, z1 = _tc1(xp, W1l, W1r, b1)
    acc1, cnt = _sc_agg(y1, src, dst, with_cnt=True)
    y2, z2 = _tc2(acc1, cnt, z1, W2l, W2r, b2)
    acc2 = _sc_agg(y2, src, dst)
    return _tc3(acc2, cnt, z2)[:N]


# speedup vs baseline: 1.5254x; 1.1801x over previous
import functools

import jax
import jax.numpy as jnp
from jax import lax
from jax.experimental import pallas as pl
from jax.experimental.pallas import tpu as pltpu
from jax.experimental.pallas import tpu_sc as plsc

N = 10000
NP = 10240
E = 320000
D = 128

NC = 2
NS = 16
NW = NC * NS
CHUNK = 80
E_PER_W = E // NW
N_CHUNKS = E_PER_W // CHUNK
N_CHUNKS_P = 126
N_CHUNKS_A = 128
PH = 8
ROWS_PER_TILE = NP // NS
ZB = 40


def _mm_xt(a, w):
    return lax.dot_general(a, w, (((1,), (1,)), ((), ())),
                           preferred_element_type=jnp.float32)


def _tc1_body(x_ref, wl_ref, wr_ref, b_ref, y_ref, z_ref):
    xb = x_ref[...]
    y_ref[...] = _mm_xt(xb, wl_ref[...])
    z_ref[...] = _mm_xt(xb, wr_ref[...]) + b_ref[...]


def _tc2_body(acc_ref, cnt_ref, z_ref, wl_ref, wr_ref, b_ref, y_ref, z2_ref):
    a = acc_ref[0, :, :] + acc_ref[1, :, :]
    c = cnt_ref[0, :, 0:1] + cnt_ref[1, :, 0:1]
    inv = 1.0 / jnp.maximum(c, 1.0)
    h = jnp.maximum(a * inv + z_ref[...], 0.0)
    y_ref[...] = _mm_xt(h, wl_ref[...])
    z2_ref[...] = _mm_xt(h, wr_ref[...]) + b_ref[...]


def _tc3_body(acc_ref, cnt_ref, z_ref, out_ref):
    a = acc_ref[0, :, :] + acc_ref[1, :, :]
    c = cnt_ref[0, :, 0:1] + cnt_ref[1, :, 0:1]
    inv = 1.0 / jnp.maximum(c, 1.0)
    out_ref[...] = a * inv + z_ref[...]


_BM = 1024


def _tc1(x, wl, wr, b):
    grid = (NP // _BM,)
    return pl.pallas_call(
        _tc1_body,
        grid=grid,
        in_specs=[
            pl.BlockSpec((_BM, D), lambda i: (i, 0)),
            pl.BlockSpec((D, D), lambda i: (0, 0)),
            pl.BlockSpec((D, D), lambda i: (0, 0)),
            pl.BlockSpec((1, D), lambda i: (0, 0)),
        ],
        out_specs=[
            pl.BlockSpec((_BM, D), lambda i: (i, 0)),
            pl.BlockSpec((_BM, D), lambda i: (i, 0)),
        ],
        out_shape=[
            jax.ShapeDtypeStruct((NP, D), jnp.float32),
            jax.ShapeDtypeStruct((NP, D), jnp.float32),
        ],
    )(x, wl, wr, b.reshape(1, D))


def _tc2(acc, cnt, z, wl, wr, b):
    grid = (NP // _BM,)
    return pl.pallas_call(
        _tc2_body,
        grid=grid,
        in_specs=[
            pl.BlockSpec((NC, _BM, D), lambda i: (0, i, 0)),
            pl.BlockSpec((NC, _BM, D), lambda i: (0, i, 0)),
            pl.BlockSpec((_BM, D), lambda i: (i, 0)),
            pl.BlockSpec((D, D), lambda i: (0, 0)),
            pl.BlockSpec((D, D), lambda i: (0, 0)),
            pl.BlockSpec((1, D), lambda i: (0, 0)),
        ],
        out_specs=[
            pl.BlockSpec((_BM, D), lambda i: (i, 0)),
            pl.BlockSpec((_BM, D), lambda i: (i, 0)),
        ],
        out_shape=[
            jax.ShapeDtypeStruct((NP, D), jnp.float32),
            jax.ShapeDtypeStruct((NP, D), jnp.float32),
        ],
    )(acc, cnt, z, wl, wr, b.reshape(1, D))


def _tc3(acc, cnt, z):
    grid = (NP // _BM,)
    return pl.pallas_call(
        _tc3_body,
        grid=grid,
        in_specs=[
            pl.BlockSpec((NC, _BM, D), lambda i: (0, i, 0)),
            pl.BlockSpec((NC, _BM, D), lambda i: (0, i, 0)),
            pl.BlockSpec((_BM, D), lambda i: (i, 0)),
        ],
        out_specs=pl.BlockSpec((_BM, D), lambda i: (i, 0)),
        out_shape=jax.ShapeDtypeStruct((NP, D), jnp.float32),
    )(acc, cnt, z)


def _zero_fill(ref, rows, cols):
    z16 = jnp.zeros((16,), jnp.float32)
    for r in range(rows):
        for c in range(cols // 16):
            ref[r, pl.ds(c * 16, 16)] = z16


def _zero_fill3(ref, b, rows, cols):
    z16 = jnp.zeros((16,), jnp.float32)
    for r in range(rows):
        for c in range(cols // 16):
            ref[b, r, pl.ds(c * 16, 16)] = z16


def _ones_fill3(ref, b, rows, cols):
    o16 = jnp.ones((16,), jnp.float32)
    for r in range(rows):
        for c in range(cols // 16):
            ref[b, r, pl.ds(c * 16, 16)] = o16


def _zero_acc(rows_v, acc_sh, base_n):
    _zero_fill3(rows_v, 0, CHUNK, D)
    for k in range(ROWS_PER_TILE // CHUNK):
        pltpu.sync_copy(rows_v.at[0], acc_sh.at[pl.ds(base_n + k * CHUNK, CHUNK)])


def _agg_pipeline(y_hbm, sd_hbm, acc_sh, sdA, sdB, rows_v, semA, semB, wid):
    pltpu.sync_copy(sd_hbm.at[wid, 0], sdA)
    g0 = pltpu.async_copy(y_hbm.at[sdA.at[0]], rows_v.at[0], semA)
    g0.wait()

    def step(j, carry):
        c0 = 2 * j + 1
        c1 = 2 * j + 2
        pltpu.sync_copy(sd_hbm.at[wid, c0], sdB)
        gB = pltpu.async_copy(y_hbm.at[sdB.at[0]], rows_v.at[1], semB)
        pltpu.sync_copy(rows_v.at[0], acc_sh.at[sdA.at[1]], add=True)
        pltpu.sync_copy(sd_hbm.at[wid, c1], sdA)
        gA = pltpu.async_copy(y_hbm.at[sdA.at[0]], rows_v.at[0], semA)
        gB.wait()
        pltpu.sync_copy(rows_v.at[1], acc_sh.at[sdB.at[1]], add=True)
        gA.wait()
        return carry

    lax.fori_loop(0, (N_CHUNKS - 1) // 2, step, 0)
    pltpu.sync_copy(rows_v.at[0], acc_sh.at[sdA.at[1]], add=True)


def _sc_agg_body(with_cnt, y_hbm, sd_hbm, *rest):
    if with_cnt:
        (acc_out, cnt_out, acc_sh, sdA, sdB, rows_v, semA, semB) = rest
    else:
        (acc_out, acc_sh, sdA, sdB, rows_v, semA, semB) = rest
        cnt_out = None
    cid = lax.axis_index("c")
    sid = lax.axis_index("s")
    wid = cid * NS + sid
    base_n = sid * ROWS_PER_TILE

    if with_cnt:
        _zero_acc(rows_v, acc_sh, base_n)
        _ones_fill3(rows_v, 1, CHUNK, D)
        plsc.subcore_barrier()

        iA = pltpu.async_copy(sd_hbm.at[wid, 0], sdA, semA)
        iA.wait()

        def cstep(j, carry):
            c0 = 2 * j + 1
            c1 = 2 * j + 2
            iB = pltpu.async_copy(sd_hbm.at[wid, c0], sdB, semB)
            pltpu.sync_copy(rows_v.at[1], acc_sh.at[sdA.at[1]], add=True)
            iA2 = pltpu.async_copy(sd_hbm.at[wid, c1], sdA, semA)
            iB.wait()
            pltpu.sync_copy(rows_v.at[1], acc_sh.at[sdB.at[1]], add=True)
            iA2.wait()
            return carry

        lax.fori_loop(0, (N_CHUNKS - 1) // 2, cstep, 0)
        pltpu.sync_copy(rows_v.at[1], acc_sh.at[sdA.at[1]], add=True)
        plsc.subcore_barrier()
        pltpu.sync_copy(acc_sh.at[pl.ds(base_n, ROWS_PER_TILE)],
                        cnt_out.at[cid, pl.ds(base_n, ROWS_PER_TILE)])
        plsc.subcore_barrier()

    _zero_acc(rows_v, acc_sh, base_n)
    plsc.subcore_barrier()
    _agg_pipeline(y_hbm, sd_hbm, acc_sh, sdA, sdB, rows_v, semA, semB, wid)
    plsc.subcore_barrier()
    pltpu.sync_copy(acc_sh.at[pl.ds(base_n, ROWS_PER_TILE)],
                    acc_out.at[cid, pl.ds(base_n, ROWS_PER_TILE)])


def _sc_agg(y, sd3, with_cnt=False):
    mesh = plsc.VectorSubcoreMesh(core_axis_name="c", subcore_axis_name="s")
    out_type = [jax.ShapeDtypeStruct((NC, NP, D), jnp.float32)]
    if with_cnt:
        out_type.append(jax.ShapeDtypeStruct((NC, NP, D), jnp.float32))
    fn = pl.kernel(
        functools.partial(_sc_agg_body, with_cnt),
        out_type=tuple(out_type),
        mesh=mesh,
        scratch_types=[
            pltpu.VMEM_SHARED((NP, D), jnp.float32),
            pltpu.VMEM((2, CHUNK), jnp.int32),
            pltpu.VMEM((2, CHUNK), jnp.int32),
            pltpu.VMEM((2, CHUNK, D), jnp.float32),
            pltpu.SemaphoreType.DMA,
            pltpu.SemaphoreType.DMA,
        ],
    )
    out = fn(y, sd3)
    return out if with_cnt else out[0]


def kernel(x, edge_index, W1l, W1r, b1, W2l, W2r, b2):
    src = edge_index[0].astype(jnp.int32).reshape(NW, N_CHUNKS, 1, CHUNK)
    dst = edge_index[1].astype(jnp.int32).reshape(NW, N_CHUNKS, 1, CHUNK)
    sd = jnp.concatenate([src, dst], axis=2)
    xp = jnp.pad(x, ((0, NP - N), (0, 0)))

    y1, z1 = _tc1(xp, W1l, W1r, b1)
    acc1, cnt = _sc_agg(y1, sd, with_cnt=True)
    y2, z2 = _tc2(acc1, cnt, z1, W2l, W2r, b2)
    acc2 = _sc_agg(y2, sd)
    return _tc3(acc2, cnt, z2)[:N]


# speedup vs baseline: 1.5268x; 1.0009x over previous
import functools

import jax
import jax.numpy as jnp
from jax import lax
from jax.experimental import pallas as pl
from jax.experimental.pallas import tpu as pltpu
from jax.experimental.pallas import tpu_sc as plsc

N = 10000
NP = 10240
E = 320000
D = 128

NC = 2
NS = 16
NW = NC * NS
CHUNK = 80
E_PER_W = E // NW
N_CHUNKS = E_PER_W // CHUNK
ROWS_PER_TILE = NP // NS


def _mm_xt(a, w):
    return lax.dot_general(a, w, (((1,), (1,)), ((), ())),
                           preferred_element_type=jnp.float32)


def _tc1_body(x_ref, wl_ref, wr_ref, b_ref, y_ref, z_ref):
    xb = x_ref[...]
    y_ref[...] = _mm_xt(xb, wl_ref[...])
    z_ref[...] = _mm_xt(xb, wr_ref[...]) + b_ref[...]


def _tc2_body(acc_ref, cnt_ref, z_ref, wl_ref, wr_ref, b_ref, y_ref, z2_ref):
    a = acc_ref[0, :, :] + acc_ref[1, :, :]
    c = cnt_ref[0, :, 0:1] + cnt_ref[1, :, 0:1]
    inv = 1.0 / jnp.maximum(c, 1.0)
    h = jnp.maximum(a * inv + z_ref[...], 0.0)
    y_ref[...] = _mm_xt(h, wl_ref[...])
    z2_ref[...] = _mm_xt(h, wr_ref[...]) + b_ref[...]


def _tc3_body(acc_ref, cnt_ref, z_ref, out_ref):
    a = acc_ref[0, :, :] + acc_ref[1, :, :]
    c = cnt_ref[0, :, 0:1] + cnt_ref[1, :, 0:1]
    inv = 1.0 / jnp.maximum(c, 1.0)
    out_ref[...] = a * inv + z_ref[...]


_BM = 1024


def _tc1(x, wl, wr, b):
    grid = (NP // _BM,)
    return pl.pallas_call(
        _tc1_body,
        grid=grid,
        in_specs=[
            pl.BlockSpec((_BM, D), lambda i: (i, 0)),
            pl.BlockSpec((D, D), lambda i: (0, 0)),
            pl.BlockSpec((D, D), lambda i: (0, 0)),
            pl.BlockSpec((1, D), lambda i: (0, 0)),
        ],
        out_specs=[
            pl.BlockSpec((_BM, D), lambda i: (i, 0)),
            pl.BlockSpec((_BM, D), lambda i: (i, 0)),
        ],
        out_shape=[
            jax.ShapeDtypeStruct((NP, D), jnp.float32),
            jax.ShapeDtypeStruct((NP, D), jnp.float32),
        ],
    )(x, wl, wr, b.reshape(1, D))


def _tc2(acc, cnt, z, wl, wr, b):
    grid = (NP // _BM,)
    return pl.pallas_call(
        _tc2_body,
        grid=grid,
        in_specs=[
            pl.BlockSpec((NC, _BM, D), lambda i: (0, i, 0)),
            pl.BlockSpec((NC, _BM, D), lambda i: (0, i, 0)),
            pl.BlockSpec((_BM, D), lambda i: (i, 0)),
            pl.BlockSpec((D, D), lambda i: (0, 0)),
            pl.BlockSpec((D, D), lambda i: (0, 0)),
            pl.BlockSpec((1, D), lambda i: (0, 0)),
        ],
        out_specs=[
            pl.BlockSpec((_BM, D), lambda i: (i, 0)),
            pl.BlockSpec((_BM, D), lambda i: (i, 0)),
        ],
        out_shape=[
            jax.ShapeDtypeStruct((NP, D), jnp.float32),
            jax.ShapeDtypeStruct((NP, D), jnp.float32),
        ],
    )(acc, cnt, z, wl, wr, b.reshape(1, D))


def _tc3(acc, cnt, z):
    grid = (NP // _BM,)
    return pl.pallas_call(
        _tc3_body,
        grid=grid,
        in_specs=[
            pl.BlockSpec((NC, _BM, D), lambda i: (0, i, 0)),
            pl.BlockSpec((NC, _BM, D), lambda i: (0, i, 0)),
            pl.BlockSpec((_BM, D), lambda i: (i, 0)),
        ],
        out_specs=pl.BlockSpec((_BM, D), lambda i: (i, 0)),
        out_shape=jax.ShapeDtypeStruct((NP, D), jnp.float32),
    )(acc, cnt, z)


def _zero_fill3(ref, b, rows, cols):
    z16 = jnp.zeros((16,), jnp.float32)
    for r in range(rows):
        for c in range(cols // 16):
            ref[b, r, pl.ds(c * 16, 16)] = z16


def _ones_fill3(ref, b, rows, cols):
    o16 = jnp.ones((16,), jnp.float32)
    for r in range(rows):
        for c in range(cols // 16):
            ref[b, r, pl.ds(c * 16, 16)] = o16


def _zero_acc(rows_v, acc_sh, base_n):
    _zero_fill3(rows_v, 0, CHUNK, D)
    for k in range(ROWS_PER_TILE // CHUNK):
        pltpu.sync_copy(rows_v.at[0], acc_sh.at[pl.ds(base_n + k * CHUNK, CHUNK)])


def _agg_pipeline(y_hbm, sd_hbm, acc_sh, sdA, sdB, rows_v, semA, semB, wid):
    pltpu.sync_copy(sd_hbm.at[wid, 0], sdA)
    g0 = pltpu.async_copy(y_hbm.at[sdA.at[0]], rows_v.at[0], semA)
    g0.wait()

    def step(j, carry):
        c0 = 2 * j + 1
        c1 = 2 * j + 2
        pltpu.sync_copy(sd_hbm.at[wid, c0], sdB)
        gB = pltpu.async_copy(y_hbm.at[sdB.at[0]], rows_v.at[1], semB)
        pltpu.sync_copy(rows_v.at[0], acc_sh.at[sdA.at[1]], add=True)
        pltpu.sync_copy(sd_hbm.at[wid, c1], sdA)
        gA = pltpu.async_copy(y_hbm.at[sdA.at[0]], rows_v.at[0], semA)
        gB.wait()
        pltpu.sync_copy(rows_v.at[1], acc_sh.at[sdB.at[1]], add=True)
        gA.wait()
        return carry

    lax.fori_loop(0, (N_CHUNKS - 1) // 2, step, 0)
    pltpu.sync_copy(rows_v.at[0], acc_sh.at[sdA.at[1]], add=True)


def _sc_agg_body(with_cnt, y_hbm, sd_hbm, *rest):
    if with_cnt:
        (acc_out, cnt_out, acc_sh, sdA, sdB, rows_v, semA, semB) = rest
    else:
        (acc_out, acc_sh, sdA, sdB, rows_v, semA, semB) = rest
        cnt_out = None
    cid = lax.axis_index("c")
    sid = lax.axis_index("s")
    wid = cid * NS + sid
    base_n = sid * ROWS_PER_TILE

    if with_cnt:
        _zero_acc(rows_v, acc_sh, base_n)
        _ones_fill3(rows_v, 1, CHUNK, D)
        plsc.subcore_barrier()

        iA = pltpu.async_copy(sd_hbm.at[wid, 0], sdA, semA)
        iA.wait()

        def cstep(j, carry):
            c0 = 2 * j + 1
            c1 = 2 * j + 2
            iB = pltpu.async_copy(sd_hbm.at[wid, c0], sdB, semB)
            pltpu.sync_copy(rows_v.at[1], acc_sh.at[sdA.at[1]], add=True)
            iA2 = pltpu.async_copy(sd_hbm.at[wid, c1], sdA, semA)
            iB.wait()
            pltpu.sync_copy(rows_v.at[1], acc_sh.at[sdB.at[1]], add=True)
            iA2.wait()
            return carry

        lax.fori_loop(0, (N_CHUNKS - 1) // 2, cstep, 0)
        pltpu.sync_copy(rows_v.at[1], acc_sh.at[sdA.at[1]], add=True)
        plsc.subcore_barrier()
        pltpu.sync_copy(acc_sh.at[pl.ds(base_n, ROWS_PER_TILE)],
                        cnt_out.at[cid, pl.ds(base_n, ROWS_PER_TILE)])
        plsc.subcore_barrier()

    _zero_acc(rows_v, acc_sh, base_n)
    plsc.subcore_barrier()
    _agg_pipeline(y_hbm, sd_hbm, acc_sh, sdA, sdB, rows_v, semA, semB, wid)
    plsc.subcore_barrier()
    pltpu.sync_copy(acc_sh.at[pl.ds(base_n, ROWS_PER_TILE)],
                    acc_out.at[cid, pl.ds(base_n, ROWS_PER_TILE)])


def _sc_agg(y, sd3, with_cnt=False):
    mesh = plsc.VectorSubcoreMesh(core_axis_name="c", subcore_axis_name="s")
    out_type = [jax.ShapeDtypeStruct((NC, NP, D), jnp.float32)]
    if with_cnt:
        out_type.append(jax.ShapeDtypeStruct((NC, NP, D), jnp.float32))
    fn = pl.kernel(
        functools.partial(_sc_agg_body, with_cnt),
        out_type=tuple(out_type),
        mesh=mesh,
        scratch_types=[
            pltpu.VMEM_SHARED((NP, D), jnp.float32),
            pltpu.VMEM((2, CHUNK), jnp.int32),
            pltpu.VMEM((2, CHUNK), jnp.int32),
            pltpu.VMEM((2, CHUNK, D), jnp.float32),
            pltpu.SemaphoreType.DMA,
            pltpu.SemaphoreType.DMA,
        ],
    )
    out = fn(y, sd3)
    return out if with_cnt else out[0]


def kernel(x, edge_index, W1l, W1r, b1, W2l, W2r, b2):
    src = edge_index[0].astype(jnp.int32).reshape(NW, N_CHUNKS, 1, CHUNK)
    dst = edge_index[1].astype(jnp.int32).reshape(NW, N_CHUNKS, 1, CHUNK)
    sd = jnp.concatenate([src, dst], axis=2)
    xp = jnp.pad(x, ((0, NP - N), (0, 0)))

    y1, z1 = _tc1(xp, W1l, W1r, b1)
    acc1, cnt = _sc_agg(y1, sd, with_cnt=True)
    y2, z2 = _tc2(acc1, cnt, z1, W2l, W2r, b2)
    acc2 = _sc_agg(y2, sd)
    return _tc3(acc2, cnt, z2)[:N]
